# baseline (speedup 1.0000x reference)
import functools
import jax
import jax.numpy as jnp
from jax import lax
from jax.experimental import pallas as pl
from jax.experimental.pallas import tpu as pltpu
from jax.experimental.pallas import tpu_sc as plsc

_CAT = (0, 5, 10, 15)
_K = 101
_B, _T, _F = 1024, 20, 32

_N = _B * _T
_CHUNK = 4 * _N // 16
_RPT = _B // 8
_PIPE = 8


def _sc_kern(xq_hbm, o0, o1, o2, o3,
             xin, table, rb16, tbl2, pbuf, shared, sem):
    zero16 = jnp.zeros((16,), jnp.float32)
    one16 = jnp.ones((16,), jnp.float32)
    iota16 = lax.iota(jnp.int32, 16)
    c = lax.axis_index("c")
    s = lax.axis_index("s")

    for j in range(8):
        table[pl.ds(16 * j, 16)] = zero16

    pltpu.sync_copy(xq_hbm.at[pl.ds(s * _CHUNK, _CHUNK)], xin)

    def scat(j, carry):
        v = xin[pl.ds(16 * j, 16)]
        i = v.astype(jnp.int32) + 1
        i = jnp.where(i < 0, i + _K, i)
        i = jnp.clip(i, 0, 127)
        plsc.store_scatter(table, [i], one16)
        return carry

    lax.fori_loop(0, _CHUNK // 16, scat, 0)

    pltpu.sync_copy(table, shared.at[s])
    plsc.subcore_barrier()
    pltpu.sync_copy(shared, rb16)

    hi = (s >= 8).astype(jnp.int32)
    base = 8 * c + 4 * hi

    def red(j, carry):
        sl = pl.ds(16 * j, 16)
        acc = rb16[base, sl]
        for r in range(1, 4):
            acc = acc + rb16[base + r, sl]
        tbl2[sl] = acc
        return carry

    lax.fori_loop(0, 8, red, 0)

    idxs = []
    for m in range(7):
        idx = jnp.minimum(16 * m + iota16, 127)
        v = jnp.minimum(plsc.load_gather(tbl2, [idx]), 1.0)
        valid = (16 * m + iota16) < _K
        idxs.append((idx, v, valid))

    def fill(t, carry):
        tv = jnp.full((16,), t, jnp.int32)
        for idx, v, valid in idxs:
            plsc.store_scatter(pbuf, [tv, idx], v, mask=valid)
        return carry

    lax.fori_loop(0, _T, fill, 0)

    row0 = _RPT * (s % 8)

    for half, olo, ohi in ((0, o0, o1), (1, o2, o3)):

        @pl.when(c == half)
        def _(olo=olo, ohi=ohi):
            for which, o in ((0, olo), (1, ohi)):

                @pl.when(hi == which)
                def _(o=o):
                    def wr(i, carry):
                        cps = [
                            pltpu.make_async_copy(
                                pbuf, o.at[row0 + _PIPE * i + k], sem)
                            for k in range(_PIPE)
                        ]
                        for cp in cps:
                            cp.start()
                        for cp in cps:
                            cp.wait()
                        return carry

                    lax.fori_loop(0, _RPT // _PIPE, wr, 0)


def _sc_all(xq):
    mesh = plsc.VectorSubcoreMesh(core_axis_name="c", subcore_axis_name="s")
    kern = functools.partial(
        pl.kernel,
        out_type=[jax.ShapeDtypeStruct((_B, _T, _K), jnp.float32)] * 4,
        mesh=mesh,
        compiler_params=pltpu.CompilerParams(needs_layout_passes=False),
        scratch_types=[
            pltpu.VMEM((_CHUNK,), jnp.float32),
            pltpu.VMEM((128,), jnp.float32),
            pltpu.VMEM((16, 128), jnp.float32),
            pltpu.VMEM((128,), jnp.float32),
            pltpu.VMEM((_T, _K), jnp.float32),
            pltpu.VMEM_SHARED((16, 128), jnp.float32),
            pltpu.SemaphoreType.DMA,
        ],
    )(_sc_kern)
    return kern(xq)


def kernel(x, W, b):
    xq = jnp.concatenate([x[:, :, c].reshape(-1) for c in _CAT])
    c4 = _sc_all(xq)
    return (x, x, c4[0], c4[1], c4[2], c4[3])


# speedup vs baseline: 1.2244x; 1.2244x over previous
import jax
import jax.numpy as jnp
from jax import lax
from jax.experimental import pallas as pl
from jax.experimental.pallas import tpu as pltpu

_CAT = (0, 5, 10, 15)
_K = 101
_B, _T, _F = 1024, 20, 32
_R = (_B * _T) // 128
_BS = 256
_G = _B // _BS


def _kern(xs_ref, o0, o1, o2, o3, mask_ref):
    step = pl.program_id(0)

    @pl.when(step == 0)
    def _masks():
        li = jax.lax.broadcasted_iota(jnp.int32, (1, 128), 1)
        for f in range(4):
            v = xs_ref[f * _R:(f + 1) * _R, :]
            i = v.astype(jnp.int32) + 1
            i = jnp.where(i < 0, i + _K, i)
            i = jnp.clip(i, 0, 127)
            sh = jnp.left_shift(jnp.int32(1), i & 31)
            w = i >> 5
            wvecs = []
            for word in range(4):
                a = jnp.where(w == word, sh, 0)
                n = _R
                while n > 8:
                    h = (n + 1) // 2
                    a = a[0:n - h] | a[h:n]
                    n = h
                acc = a[0:1]
                for r in range(1, n):
                    acc = acc | a[r:r + 1]
                for lsh in (1, 2, 4, 8, 16, 32, 64):
                    acc = acc | pltpu.roll(acc, lsh, 1)
                wvecs.append(acc)
            wv = jnp.where(li < 32, wvecs[0],
                           jnp.where(li < 64, wvecs[1],
                                     jnp.where(li < 96, wvecs[2], wvecs[3])))
            bit = (jnp.right_shift(wv, li & 31)) & 1
            mask_ref[f:f + 1, :] = bit.astype(jnp.float32)

    for f, o in enumerate((o0, o1, o2, o3)):
        m = mask_ref[f:f + 1, 0:_K]
        o[...] = jnp.broadcast_to(m.reshape(1, 1, _K), (_BS, _T, _K))


def kernel(x, W, b):
    xs = jnp.concatenate(
        [x[:, :, c].reshape(_R, 128) for c in _CAT], axis=0)
    c = pl.pallas_call(
        _kern,
        grid=(_G,),
        in_specs=[pl.BlockSpec((4 * _R, 128), lambda i: (0, 0))],
        out_specs=[pl.BlockSpec((_BS, _T, _K), lambda i: (i, 0, 0))] * 4,
        out_shape=[jax.ShapeDtypeStruct((_B, _T, _K), jnp.float32)] * 4,
        scratch_shapes=[pltpu.VMEM((8, 128), jnp.float32)],
    )(xs)
    return (x, x, c[0], c[1], c[2], c[3])
